# baseline (speedup 1.0000x reference)
import functools

import jax
import jax.numpy as jnp
from jax.experimental import pallas as pl


BATCH = 16384
NG = 64
DIM = 64
BB = 1024
GC = 8
CHUNK = GC * DIM


GH = 8
GL = 8
WIDE = GH * DIM


def _body(k_ref, eps_ref, w_ref, mu_ref, out_ref):
    eps = eps_ref[...]
    kb = k_ref[...]
    kh = kb >> 3
    kl = kb & 7
    eps_t = jnp.concatenate([eps] * GH, axis=1)
    c1 = jax.lax.broadcasted_iota(jnp.int32, (BB, WIDE), 1)
    e1 = jnp.where((c1 >> 6) == kh, eps_t, 0.0).astype(jnp.bfloat16)
    y = jnp.dot(e1, w_ref[...], preferred_element_type=jnp.float32)
    y = jnp.where((kl & 4) != 0, y[:, 256:512], y[:, 0:256])
    y = jnp.where((kl & 2) != 0, y[:, 128:256], y[:, 0:128])
    y = jnp.where((kl & 1) != 0, y[:, 64:128], y[:, 0:64])
    g64 = jax.lax.broadcasted_iota(jnp.int32, (BB, NG), 1)
    oh = (g64 == kb).astype(jnp.float32)
    out_ref[...] = y + jnp.dot(oh, mu_ref[...],
                               preferred_element_type=jnp.float32)


@jax.jit
def _run(k_col, eps, w, mu):
    grid = (BATCH // BB,)
    return pl.pallas_call(
        _body,
        grid=grid,
        in_specs=[
            pl.BlockSpec((BB, 1), lambda i: (i, 0)),
            pl.BlockSpec((BB, DIM), lambda i: (i, 0)),
            pl.BlockSpec((WIDE, WIDE), lambda i: (0, 0)),
            pl.BlockSpec((NG, DIM), lambda i: (0, 0)),
        ],
        out_specs=pl.BlockSpec((BB, DIM), lambda i: (i, 0)),
        out_shape=jax.ShapeDtypeStruct((BATCH, DIM), jnp.float32),
    )(k_col, eps, w, mu)


def kernel(batch_size, k, epsilon, mu, A):
    k_col = k.astype(jnp.int32).reshape(BATCH, 1)
    w = (A.reshape(GH, GL, DIM, DIM).transpose(0, 3, 1, 2)
         .reshape(WIDE, WIDE).astype(jnp.bfloat16))
    return _run(k_col, epsilon, w, mu)


# speedup vs baseline: 1.0818x; 1.0818x over previous
import functools

import jax
import jax.numpy as jnp
from jax.experimental import pallas as pl


BATCH = 16384
NG = 64
DIM = 64
BB = 2048
GC = 8
CHUNK = GC * DIM


GH = 8
GL = 8
WIDE = GH * DIM


def _body(k_ref, eps_ref, w_ref, mu_ref, out_ref):
    eps = eps_ref[...]
    kb = k_ref[...]
    kh = kb >> 3
    kl = kb & 7
    eps_t = jnp.concatenate([eps] * GH, axis=1)
    c1 = jax.lax.broadcasted_iota(jnp.int32, (BB, WIDE), 1)
    e1 = jnp.where((c1 >> 6) == kh, eps_t, 0.0)
    y = jnp.dot(e1, w_ref[...], preferred_element_type=jnp.float32)
    y = jnp.where((kl & 4) != 0, y[:, 256:512], y[:, 0:256])
    y = jnp.where((kl & 2) != 0, y[:, 128:256], y[:, 0:128])
    y = jnp.where((kl & 1) != 0, y[:, 64:128], y[:, 0:64])
    g64 = jax.lax.broadcasted_iota(jnp.int32, (BB, NG), 1)
    oh = (g64 == kb).astype(jnp.float32)
    out_ref[...] = y + jnp.dot(oh, mu_ref[...],
                               preferred_element_type=jnp.float32)


@jax.jit
def _run(k_col, eps, w, mu):
    grid = (BATCH // BB,)
    return pl.pallas_call(
        _body,
        grid=grid,
        in_specs=[
            pl.BlockSpec((BB, 1), lambda i: (i, 0)),
            pl.BlockSpec((BB, DIM), lambda i: (i, 0)),
            pl.BlockSpec((WIDE, WIDE), lambda i: (0, 0)),
            pl.BlockSpec((NG, DIM), lambda i: (0, 0)),
        ],
        out_specs=pl.BlockSpec((BB, DIM), lambda i: (i, 0)),
        out_shape=jax.ShapeDtypeStruct((BATCH, DIM), jnp.float32),
    )(k_col, eps, w, mu)


def kernel(batch_size, k, epsilon, mu, A):
    k_col = k.astype(jnp.int32).reshape(BATCH, 1)
    w = A.reshape(GH, GL, DIM, DIM).transpose(0, 3, 1, 2).reshape(WIDE, WIDE)
    return _run(k_col, epsilon, w, mu)
